# baseline (speedup 1.0000x reference)
import functools

import jax
import jax.numpy as jnp
from jax import lax
from jax.experimental import pallas as pl
from jax.experimental.pallas import tpu as pltpu
from jax.experimental.pallas import tpu_sc as plsc

N_NODES = 100000
NODE_SIZE = 128
EDGE_SIZE = 16
OUT_SIZE = 128
GROUP = 25000
DEGS = (1, 2, 4, 8)
N_EDGES = 375000
EPS = 1e-5

NC = 2
NS = 16
NW = NC * NS
LANES = 16
BLK = 128


def _ceil_div(a, b):
    return (a + b - 1) // b


NBLKS = tuple(_ceil_div(GROUP * d, BLK) for d in DEGS)
PERS = tuple(_ceil_div(nb, NW) for nb in NBLKS)
MAXPER = max(PERS)


def _make_sc_gather_body(width):
    def body(table_hbm, i1, i2, i3, i4, sum_hbm,
             idx_v, rows, outb, gsem, osem):
        wid = lax.axis_index("s") * NC + lax.axis_index("c")
        idx_all = (i1, i2, i3, i4)
        for b, d in enumerate(DEGS):
            idx_hbm = idx_all[b]
            per = PERS[b]
            bd = BLK // d
            span = per * BLK

            s0 = pl.multiple_of(
                jnp.minimum(wid * span, GROUP * d - span), 8)
            pltpu.sync_copy(idx_hbm.at[pl.ds(s0, span)],
                            idx_v.at[pl.ds(0, span)])
            o0 = s0 // d

            def issue_g(j, p):
                pltpu.async_copy(
                    table_hbm.at[idx_v.at[pl.ds(j * BLK, BLK)]],
                    rows.at[p], gsem.at[p])

            def wait_g(p):
                pltpu.make_async_copy(
                    table_hbm.at[idx_v.at[pl.ds(0, BLK)]],
                    rows.at[p], gsem.at[p]).wait()

            def acc(j, p, d=d, bd=bd):
                if d == 1:
                    def cp_row(r, _):
                        for c in range(width // LANES):
                            outb[p, r, pl.ds(c * LANES, LANES)] = (
                                rows[p, r, pl.ds(c * LANES, LANES)])
                        return 0
                    lax.fori_loop(0, bd, cp_row, 0, unroll=4)
                else:
                    def acc_row(r, _):
                        base = r * d
                        for c in range(width // LANES):
                            s = rows[p, base, pl.ds(c * LANES, LANES)]
                            for k in range(1, d):
                                s = s + rows[p, base + k,
                                             pl.ds(c * LANES, LANES)]
                            outb[p, r, pl.ds(c * LANES, LANES)] = s
                        return 0
                    lax.fori_loop(0, bd, acc_row, 0,
                                  unroll=max(1, 8 // d))

            def issue_o(j, p, b=b, bd=bd, o0=o0):
                obase = pl.multiple_of(o0 + j * bd, 8)
                pltpu.async_copy(
                    outb.at[p, pl.ds(0, bd)],
                    sum_hbm.at[b, pl.ds(obase, bd)], osem.at[p])

            def wait_o(p, b=b, bd=bd):
                pltpu.make_async_copy(
                    outb.at[p, pl.ds(0, bd)],
                    sum_hbm.at[b, pl.ds(0, bd)], osem.at[p]).wait()

            issue_g(0, 0)
            issue_g(1, 1)
            wait_g(0)
            acc(0, 0)
            issue_o(0, 0)
            issue_g(2, 0)
            wait_g(1)
            acc(1, 1)
            issue_o(1, 1)

            def pair(i, _):
                j = 2 * i + 2
                issue_g(j + 1, 1)
                wait_g(0)
                wait_o(0)
                acc(j, 0)
                issue_o(j, 0)
                issue_g(j + 2, 0)
                wait_g(1)
                wait_o(1)
                acc(j + 1, 1)
                issue_o(j + 1, 1)
                return 0

            lax.fori_loop(0, (per - 3) // 2, pair, 0, unroll=False)

            wait_g(0)
            wait_o(0)
            acc(per - 1, 0)
            issue_o(per - 1, 0)
            wait_o(0)
            wait_o(1)

    return body


def _sc_gather_sums(table, idx4, width, tc_tiling):
    mesh = plsc.VectorSubcoreMesh(core_axis_name="c", subcore_axis_name="s")
    out_type = jax.ShapeDtypeStruct((4, GROUP, width), jnp.float32)
    scratch = [
        pltpu.VMEM((MAXPER * BLK,), jnp.int32),
        pltpu.VMEM((2, BLK, width), jnp.float32),
        pltpu.VMEM((2, BLK, width), jnp.float32),
        pltpu.SemaphoreType.DMA((2,)),
        pltpu.SemaphoreType.DMA((2,)),
    ]
    fn = pl.kernel(_make_sc_gather_body(width), out_type=out_type, mesh=mesh,
                   scratch_types=scratch,
                   compiler_params=pltpu.CompilerParams(
                       use_tc_tiling_on_sc=tc_tiling))
    return fn(table, *idx4)


R1 = 1000
NB1 = 4 * GROUP // R1
BPB = GROUP // R1


R0 = 5000
NB0 = 4 * GROUP // R0


def _tc_self_body(node_ref, wself_ref, bias_ref, out_ref):
    out_ref[...] = jnp.dot(node_ref[...], wself_ref[...],
                           preferred_element_type=jnp.float32) + bias_ref[...]


def _tc_self(node_repr, W_self, bias):
    return pl.pallas_call(
        _tc_self_body,
        grid=(NB0,),
        in_specs=[
            pl.BlockSpec((R0, NODE_SIZE), lambda g: (g, 0)),
            pl.BlockSpec((NODE_SIZE, OUT_SIZE), lambda g: (0, 0)),
            pl.BlockSpec((1, OUT_SIZE), lambda g: (0, 0)),
        ],
        out_specs=pl.BlockSpec((R0, OUT_SIZE), lambda g: (g, 0)),
        out_shape=jax.ShapeDtypeStruct((4 * GROUP, OUT_SIZE), jnp.float32),
    )(node_repr, W_self, bias)


def _tc_fused_body(self_ref, nsum_ref, esum_ref, wn_ref, we_ref,
                   out_ref, act_scr, s1_scr, s2_scr, stat_scr):
    p = pl.program_id(0)
    g = pl.program_id(1)

    @pl.when(p == 0)
    def _():
        act = self_ref[...]
        act = act + jnp.dot(nsum_ref[0], wn_ref[0],
                            preferred_element_type=jnp.float32)
        act = act + jnp.dot(esum_ref[0], we_ref[0],
                            preferred_element_type=jnp.float32)
        act_scr[pl.ds(g * R1, R1), :] = act
        m1 = jnp.sum(act.reshape(R1 // 8, 8, OUT_SIZE), axis=0)
        m2 = jnp.sum((act * act).reshape(R1 // 8, 8, OUT_SIZE), axis=0)

        @pl.when(g == 0)
        def _():
            s1_scr[...] = m1
            s2_scr[...] = m2

        @pl.when(g > 0)
        def _():
            s1_scr[...] += m1
            s2_scr[...] += m2

    @pl.when(p == 1)
    def _():
        @pl.when(g == 0)
        def _():
            n = jnp.float32(4 * GROUP)
            mu = jnp.sum(s1_scr[...], axis=0, keepdims=True) / n
            var = jnp.sum(s2_scr[...], axis=0, keepdims=True) / n - mu * mu
            stat_scr[0:1, :] = mu
            stat_scr[1:2, :] = lax.rsqrt(var + EPS)

        act = act_scr[pl.ds(g * R1, R1), :]
        out_ref[...] = jnp.maximum(
            (act - stat_scr[0:1, :]) * stat_scr[1:2, :], 0.0)


def _tc_fused(self0, nsum, esum, wn_stack, we_stack):
    return pl.pallas_call(
        _tc_fused_body,
        grid=(2, NB1),
        in_specs=[
            pl.BlockSpec((R1, OUT_SIZE),
                         lambda p, g: (jnp.where(p == 0, g, NB1 - 1), 0)),
            pl.BlockSpec((1, R1, NODE_SIZE),
                         lambda p, g: (jnp.where(p == 0, g // BPB, 3),
                                       jnp.where(p == 0, g % BPB, BPB - 1),
                                       0)),
            pl.BlockSpec((1, R1, EDGE_SIZE),
                         lambda p, g: (jnp.where(p == 0, g // BPB, 3),
                                       jnp.where(p == 0, g % BPB, BPB - 1),
                                       0)),
            pl.BlockSpec((1, NODE_SIZE, OUT_SIZE),
                         lambda p, g: (jnp.where(p == 0, g // BPB, 3), 0, 0)),
            pl.BlockSpec((1, EDGE_SIZE, OUT_SIZE),
                         lambda p, g: (jnp.where(p == 0, g // BPB, 3), 0, 0)),
        ],
        out_specs=pl.BlockSpec((R1, OUT_SIZE),
                               lambda p, g: (jnp.where(p == 1, g, 0), 0)),
        out_shape=jax.ShapeDtypeStruct((4 * GROUP, OUT_SIZE), jnp.float32),
        scratch_shapes=[
            pltpu.VMEM((4 * GROUP, OUT_SIZE), jnp.float32),
            pltpu.VMEM((8, OUT_SIZE), jnp.float32),
            pltpu.VMEM((8, OUT_SIZE), jnp.float32),
            pltpu.VMEM((8, OUT_SIZE), jnp.float32),
        ],
        compiler_params=pltpu.CompilerParams(
            vmem_limit_bytes=128 * 1024 * 1024),
    )(self0, nsum, esum, wn_stack, we_stack)


def kernel(node_repr, edge_repr, nb_node_d1, nb_edge_d1, nb_node_d2,
           nb_edge_d2, nb_node_d4, nb_edge_d4, nb_node_d8, nb_edge_d8,
           W_self, W_deg1, W_deg2, W_deg4, W_deg8, W_deg16, bias):
    nidx = tuple(a.reshape(-1).astype(jnp.int32)
                 for a in (nb_node_d1, nb_node_d2, nb_node_d4, nb_node_d8))
    eidx = tuple(a.reshape(-1).astype(jnp.int32)
                 for a in (nb_edge_d1, nb_edge_d2, nb_edge_d4, nb_edge_d8))
    nsum = _sc_gather_sums(node_repr, nidx, NODE_SIZE, True)
    esum = _sc_gather_sums(edge_repr, eidx, EDGE_SIZE, False)

    self0 = _tc_self(node_repr, W_self, bias)
    wn_stack = jnp.stack([W[:NODE_SIZE] for W in
                          (W_deg1, W_deg2, W_deg4, W_deg8)])
    we_stack = jnp.stack([W[NODE_SIZE:] for W in
                          (W_deg1, W_deg2, W_deg4, W_deg8)])
    return _tc_fused(self0, nsum, esum, wn_stack, we_stack)


# speedup vs baseline: 1.0060x; 1.0060x over previous
import functools

import jax
import jax.numpy as jnp
from jax import lax
from jax.experimental import pallas as pl
from jax.experimental.pallas import tpu as pltpu
from jax.experimental.pallas import tpu_sc as plsc

N_NODES = 100000
NODE_SIZE = 128
EDGE_SIZE = 16
OUT_SIZE = 128
GROUP = 25000
DEGS = (1, 2, 4, 8)
N_EDGES = 375000
EPS = 1e-5

NC = 2
NS = 16
NW = NC * NS
LANES = 16
BLK = 128


def _ceil_div(a, b):
    return (a + b - 1) // b


NBLKS = tuple(_ceil_div(GROUP * d, BLK) for d in DEGS)
PERS = tuple(_ceil_div(nb, NW) for nb in NBLKS)
MAXPER = max(PERS)


def _make_sc_gather_body(width):
    def body(table_hbm, i1, i2, i3, i4, sum_hbm,
             idx_v, rows, outb, gsem, osem):
        wid = lax.axis_index("s") * NC + lax.axis_index("c")
        idx_all = (i1, i2, i3, i4)
        for b, d in enumerate(DEGS):
            idx_hbm = idx_all[b]
            per = PERS[b]
            bd = BLK // d
            span = per * BLK

            s0 = pl.multiple_of(
                jnp.minimum(wid * span, GROUP * d - span), 8)
            pltpu.sync_copy(idx_hbm.at[pl.ds(s0, span)],
                            idx_v.at[pl.ds(0, span)])
            o0 = s0 // d

            def issue_g(j, p):
                pltpu.async_copy(
                    table_hbm.at[idx_v.at[pl.ds(j * BLK, BLK)]],
                    rows.at[p], gsem.at[p])

            def wait_g(p):
                pltpu.make_async_copy(
                    table_hbm.at[idx_v.at[pl.ds(0, BLK)]],
                    rows.at[p], gsem.at[p]).wait()

            def acc(j, p, d=d, bd=bd):
                if d == 1:
                    def cp_row(r, _):
                        for c in range(width // LANES):
                            outb[p, r, pl.ds(c * LANES, LANES)] = (
                                rows[p, r, pl.ds(c * LANES, LANES)])
                        return 0
                    lax.fori_loop(0, bd, cp_row, 0, unroll=4)
                else:
                    def acc_row(r, _):
                        base = r * d
                        for c in range(width // LANES):
                            s = rows[p, base, pl.ds(c * LANES, LANES)]
                            for k in range(1, d):
                                s = s + rows[p, base + k,
                                             pl.ds(c * LANES, LANES)]
                            outb[p, r, pl.ds(c * LANES, LANES)] = s
                        return 0
                    lax.fori_loop(0, bd, acc_row, 0,
                                  unroll=max(1, 8 // d))

            def issue_o(j, p, b=b, bd=bd, o0=o0):
                obase = pl.multiple_of(o0 + j * bd, 8)
                pltpu.async_copy(
                    outb.at[p, pl.ds(0, bd)],
                    sum_hbm.at[b, pl.ds(obase, bd)], osem.at[p])

            def wait_o(p, b=b, bd=bd):
                pltpu.make_async_copy(
                    outb.at[p, pl.ds(0, bd)],
                    sum_hbm.at[b, pl.ds(0, bd)], osem.at[p]).wait()

            issue_g(0, 0)
            issue_g(1, 1)
            wait_g(0)
            acc(0, 0)
            issue_o(0, 0)
            issue_g(2, 0)
            wait_g(1)
            acc(1, 1)
            issue_o(1, 1)

            def pair(i, _):
                j = 2 * i + 2
                issue_g(j + 1, 1)
                wait_g(0)
                wait_o(0)
                acc(j, 0)
                issue_o(j, 0)
                issue_g(j + 2, 0)
                wait_g(1)
                wait_o(1)
                acc(j + 1, 1)
                issue_o(j + 1, 1)
                return 0

            lax.fori_loop(0, (per - 3) // 2, pair, 0, unroll=False)

            wait_g(0)
            wait_o(0)
            acc(per - 1, 0)
            issue_o(per - 1, 0)
            wait_o(0)
            wait_o(1)

    return body


def _sc_gather_sums(table, idx4, width, tc_tiling):
    mesh = plsc.VectorSubcoreMesh(core_axis_name="c", subcore_axis_name="s")
    out_type = jax.ShapeDtypeStruct((4, GROUP, width), jnp.float32)
    scratch = [
        pltpu.VMEM((MAXPER * BLK,), jnp.int32),
        pltpu.VMEM((2, BLK, width), jnp.float32),
        pltpu.VMEM((2, BLK, width), jnp.float32),
        pltpu.SemaphoreType.DMA((2,)),
        pltpu.SemaphoreType.DMA((2,)),
    ]
    fn = pl.kernel(_make_sc_gather_body(width), out_type=out_type, mesh=mesh,
                   scratch_types=scratch,
                   compiler_params=pltpu.CompilerParams(
                       use_tc_tiling_on_sc=tc_tiling))
    return fn(table, *idx4)


R1 = 1000
NB1 = 4 * GROUP // R1
BPB = GROUP // R1


def _tc_fused_body(node_ref, nsum_ref, esum_ref, wself_ref, wn_ref, we_ref,
                   bias_ref, out_ref, act_scr, s1_scr, s2_scr, stat_scr):
    p = pl.program_id(0)
    g = pl.program_id(1)

    @pl.when(p == 0)
    def _():
        act = jnp.dot(node_ref[...], wself_ref[...],
                      preferred_element_type=jnp.float32) + bias_ref[...]
        act = act + jnp.dot(nsum_ref[0], wn_ref[0],
                            preferred_element_type=jnp.float32)
        act = act + jnp.dot(esum_ref[0], we_ref[0],
                            preferred_element_type=jnp.float32)
        act_scr[pl.ds(g * R1, R1), :] = act
        m1 = jnp.sum(act.reshape(R1 // 8, 8, OUT_SIZE), axis=0)
        m2 = jnp.sum((act * act).reshape(R1 // 8, 8, OUT_SIZE), axis=0)

        @pl.when(g == 0)
        def _():
            s1_scr[...] = m1
            s2_scr[...] = m2

        @pl.when(g > 0)
        def _():
            s1_scr[...] += m1
            s2_scr[...] += m2

    @pl.when(p == 1)
    def _():
        @pl.when(g == 0)
        def _():
            n = jnp.float32(4 * GROUP)
            mu = jnp.sum(s1_scr[...], axis=0, keepdims=True) / n
            var = jnp.sum(s2_scr[...], axis=0, keepdims=True) / n - mu * mu
            stat_scr[0:1, :] = mu
            stat_scr[1:2, :] = lax.rsqrt(var + EPS)

        act = act_scr[pl.ds(g * R1, R1), :]
        out_ref[...] = jnp.maximum(
            (act - stat_scr[0:1, :]) * stat_scr[1:2, :], 0.0)


def _tc_fused(node_repr, nsum, esum, W_self, wn_stack, we_stack, bias):
    return pl.pallas_call(
        _tc_fused_body,
        grid=(2, NB1),
        in_specs=[
            pl.BlockSpec((R1, NODE_SIZE),
                         lambda p, g: (jnp.where(p == 0, g, NB1 - 1), 0)),
            pl.BlockSpec((1, R1, NODE_SIZE),
                         lambda p, g: (jnp.where(p == 0, g // BPB, 3),
                                       jnp.where(p == 0, g % BPB, BPB - 1),
                                       0)),
            pl.BlockSpec((1, R1, EDGE_SIZE),
                         lambda p, g: (jnp.where(p == 0, g // BPB, 3),
                                       jnp.where(p == 0, g % BPB, BPB - 1),
                                       0)),
            pl.BlockSpec((NODE_SIZE, OUT_SIZE), lambda p, g: (0, 0)),
            pl.BlockSpec((1, NODE_SIZE, OUT_SIZE),
                         lambda p, g: (jnp.where(p == 0, g // BPB, 3), 0, 0)),
            pl.BlockSpec((1, EDGE_SIZE, OUT_SIZE),
                         lambda p, g: (jnp.where(p == 0, g // BPB, 3), 0, 0)),
            pl.BlockSpec((1, OUT_SIZE), lambda p, g: (0, 0)),
        ],
        out_specs=pl.BlockSpec((R1, OUT_SIZE),
                               lambda p, g: (jnp.where(p == 1, g, 0), 0)),
        out_shape=jax.ShapeDtypeStruct((4 * GROUP, OUT_SIZE), jnp.float32),
        scratch_shapes=[
            pltpu.VMEM((4 * GROUP, OUT_SIZE), jnp.float32),
            pltpu.VMEM((8, OUT_SIZE), jnp.float32),
            pltpu.VMEM((8, OUT_SIZE), jnp.float32),
            pltpu.VMEM((8, OUT_SIZE), jnp.float32),
        ],
        compiler_params=pltpu.CompilerParams(
            vmem_limit_bytes=128 * 1024 * 1024),
    )(node_repr, nsum, esum, W_self, wn_stack, we_stack, bias)


def kernel(node_repr, edge_repr, nb_node_d1, nb_edge_d1, nb_node_d2,
           nb_edge_d2, nb_node_d4, nb_edge_d4, nb_node_d8, nb_edge_d8,
           W_self, W_deg1, W_deg2, W_deg4, W_deg8, W_deg16, bias):
    nidx = tuple(a.reshape(-1).astype(jnp.int32)
                 for a in (nb_node_d1, nb_node_d2, nb_node_d4, nb_node_d8))
    eidx = tuple(a.reshape(-1).astype(jnp.int32)
                 for a in (nb_edge_d1, nb_edge_d2, nb_edge_d4, nb_edge_d8))
    nsum = _sc_gather_sums(node_repr, nidx, NODE_SIZE, True)
    esum = _sc_gather_sums(edge_repr, eidx, EDGE_SIZE, False)

    wn_stack = jnp.stack([W[:NODE_SIZE] for W in
                          (W_deg1, W_deg2, W_deg4, W_deg8)])
    we_stack = jnp.stack([W[NODE_SIZE:] for W in
                          (W_deg1, W_deg2, W_deg4, W_deg8)])
    return _tc_fused(node_repr, nsum, esum, W_self, wn_stack, we_stack, bias)


# speedup vs baseline: 1.0189x; 1.0129x over previous
import functools

import jax
import jax.numpy as jnp
from jax import lax
from jax.experimental import pallas as pl
from jax.experimental.pallas import tpu as pltpu
from jax.experimental.pallas import tpu_sc as plsc

N_NODES = 100000
NODE_SIZE = 128
EDGE_SIZE = 16
OUT_SIZE = 128
GROUP = 25000
DEGS = (1, 2, 4, 8)
N_EDGES = 375000
EPS = 1e-5

NC = 2
NS = 16
NW = NC * NS
LANES = 16
BLK = 128


def _ceil_div(a, b):
    return (a + b - 1) // b


NCHUNK = _ceil_div(_ceil_div(GROUP, BLK), NW)
SPAN = NCHUNK * BLK


def _make_sc_gather_body(width):
    def body(table_hbm, i1, i2, i3, i4, sum_hbm,
             idx_v, rows, accb, gsem, osem):
        wid = lax.axis_index("s") * NC + lax.axis_index("c")
        idx_all = (i1, i2, i3, i4)
        s0 = pl.multiple_of(jnp.minimum(wid * SPAN, GROUP - SPAN), 8)
        for b, d in enumerate(DEGS):
            idx_hbm = idx_all[b]
            for k in range(d):
                pltpu.sync_copy(idx_hbm.at[pl.ds(k * GROUP + s0, SPAN)],
                                idx_v.at[pl.ds(k * SPAN, SPAN)])

            steps = [(j, k) for j in range(NCHUNK) for k in range(d)]
            nst = len(steps)

            def issue_g(i, p):
                j, k = steps[i]
                pltpu.async_copy(
                    table_hbm.at[idx_v.at[pl.ds(k * SPAN + j * BLK, BLK)]],
                    rows.at[p], gsem.at[p])

            def wait_g(p):
                pltpu.make_async_copy(
                    table_hbm.at[idx_v.at[pl.ds(0, BLK)]],
                    rows.at[p], gsem.at[p]).wait()

            def acc(p, q, first):
                if first:
                    def cp_row(r, _):
                        for c in range(width // LANES):
                            accb[q, r, pl.ds(c * LANES, LANES)] = (
                                rows[p, r, pl.ds(c * LANES, LANES)])
                        return 0
                    lax.fori_loop(0, BLK, cp_row, 0, unroll=2)
                else:
                    def add_row(r, _):
                        for c in range(width // LANES):
                            accb[q, r, pl.ds(c * LANES, LANES)] += (
                                rows[p, r, pl.ds(c * LANES, LANES)])
                        return 0
                    lax.fori_loop(0, BLK, add_row, 0, unroll=1)

            def issue_o(j, q, b=b):
                obase = pl.multiple_of(s0 + j * BLK, 8)
                pltpu.async_copy(
                    accb.at[q, pl.ds(0, BLK)],
                    sum_hbm.at[b, pl.ds(obase, BLK)], osem.at[q])

            def wait_o(q, b=b):
                pltpu.make_async_copy(
                    accb.at[q, pl.ds(0, BLK)],
                    sum_hbm.at[b, pl.ds(0, BLK)], osem.at[q])  .wait()

            issue_g(0, 0)
            issue_g(1, 1)
            for i, (j, k) in enumerate(steps):
                p = i % 2
                q = j % 2
                wait_g(p)
                if k == 0 and j >= 2:
                    wait_o(q)
                acc(p, q, k == 0)
                if i + 2 < nst:
                    issue_g(i + 2, p)
                if k == d - 1:
                    issue_o(j, q)
            wait_o(0)
            wait_o(1)

    return body


def _sc_gather_sums(table, idx4, width, tc_tiling):
    mesh = plsc.VectorSubcoreMesh(core_axis_name="c", subcore_axis_name="s")
    out_type = jax.ShapeDtypeStruct((4, GROUP, width), jnp.float32)
    scratch = [
        pltpu.VMEM((max(DEGS) * SPAN,), jnp.int32),
        pltpu.VMEM((2, BLK, width), jnp.float32),
        pltpu.VMEM((2, BLK, width), jnp.float32),
        pltpu.SemaphoreType.DMA((2,)),
        pltpu.SemaphoreType.DMA((2,)),
    ]
    fn = pl.kernel(_make_sc_gather_body(width), out_type=out_type, mesh=mesh,
                   scratch_types=scratch,
                   compiler_params=pltpu.CompilerParams(
                       use_tc_tiling_on_sc=tc_tiling))
    return fn(table, *idx4)


R1 = 1000
NB1 = 4 * GROUP // R1
BPB = GROUP // R1


def _tc_fused_body(node_ref, nsum_ref, esum_ref, wself_ref, wn_ref, we_ref,
                   bias_ref, out_ref, act_scr, s1_scr, s2_scr, stat_scr):
    p = pl.program_id(0)
    g = pl.program_id(1)

    @pl.when(p == 0)
    def _():
        act = jnp.dot(node_ref[...], wself_ref[...],
                      preferred_element_type=jnp.float32) + bias_ref[...]
        act = act + jnp.dot(nsum_ref[0], wn_ref[0],
                            preferred_element_type=jnp.float32)
        act = act + jnp.dot(esum_ref[0], we_ref[0],
                            preferred_element_type=jnp.float32)
        act_scr[pl.ds(g * R1, R1), :] = act
        m1 = jnp.sum(act.reshape(R1 // 8, 8, OUT_SIZE), axis=0)
        m2 = jnp.sum((act * act).reshape(R1 // 8, 8, OUT_SIZE), axis=0)

        @pl.when(g == 0)
        def _():
            s1_scr[...] = m1
            s2_scr[...] = m2

        @pl.when(g > 0)
        def _():
            s1_scr[...] += m1
            s2_scr[...] += m2

    @pl.when(p == 1)
    def _():
        @pl.when(g == 0)
        def _():
            n = jnp.float32(4 * GROUP)
            mu = jnp.sum(s1_scr[...], axis=0, keepdims=True) / n
            var = jnp.sum(s2_scr[...], axis=0, keepdims=True) / n - mu * mu
            stat_scr[0:1, :] = mu
            stat_scr[1:2, :] = lax.rsqrt(var + EPS)

        act = act_scr[pl.ds(g * R1, R1), :]
        out_ref[...] = jnp.maximum(
            (act - stat_scr[0:1, :]) * stat_scr[1:2, :], 0.0)


def _tc_fused(node_repr, nsum, esum, W_self, wn_stack, we_stack, bias):
    return pl.pallas_call(
        _tc_fused_body,
        grid=(2, NB1),
        in_specs=[
            pl.BlockSpec((R1, NODE_SIZE),
                         lambda p, g: (jnp.where(p == 0, g, NB1 - 1), 0)),
            pl.BlockSpec((1, R1, NODE_SIZE),
                         lambda p, g: (jnp.where(p == 0, g // BPB, 3),
                                       jnp.where(p == 0, g % BPB, BPB - 1),
                                       0)),
            pl.BlockSpec((1, R1, EDGE_SIZE),
                         lambda p, g: (jnp.where(p == 0, g // BPB, 3),
                                       jnp.where(p == 0, g % BPB, BPB - 1),
                                       0)),
            pl.BlockSpec((NODE_SIZE, OUT_SIZE), lambda p, g: (0, 0)),
            pl.BlockSpec((1, NODE_SIZE, OUT_SIZE),
                         lambda p, g: (jnp.where(p == 0, g // BPB, 3), 0, 0)),
            pl.BlockSpec((1, EDGE_SIZE, OUT_SIZE),
                         lambda p, g: (jnp.where(p == 0, g // BPB, 3), 0, 0)),
            pl.BlockSpec((1, OUT_SIZE), lambda p, g: (0, 0)),
        ],
        out_specs=pl.BlockSpec((R1, OUT_SIZE),
                               lambda p, g: (jnp.where(p == 1, g, 0), 0)),
        out_shape=jax.ShapeDtypeStruct((4 * GROUP, OUT_SIZE), jnp.float32),
        scratch_shapes=[
            pltpu.VMEM((4 * GROUP, OUT_SIZE), jnp.float32),
            pltpu.VMEM((8, OUT_SIZE), jnp.float32),
            pltpu.VMEM((8, OUT_SIZE), jnp.float32),
            pltpu.VMEM((8, OUT_SIZE), jnp.float32),
        ],
        compiler_params=pltpu.CompilerParams(
            vmem_limit_bytes=128 * 1024 * 1024),
    )(node_repr, nsum, esum, W_self, wn_stack, we_stack, bias)


def kernel(node_repr, edge_repr, nb_node_d1, nb_edge_d1, nb_node_d2,
           nb_edge_d2, nb_node_d4, nb_edge_d4, nb_node_d8, nb_edge_d8,
           W_self, W_deg1, W_deg2, W_deg4, W_deg8, W_deg16, bias):
    nidx = tuple(a.T.reshape(-1).astype(jnp.int32)
                 for a in (nb_node_d1, nb_node_d2, nb_node_d4, nb_node_d8))
    eidx = tuple(a.T.reshape(-1).astype(jnp.int32)
                 for a in (nb_edge_d1, nb_edge_d2, nb_edge_d4, nb_edge_d8))
    nsum = _sc_gather_sums(node_repr, nidx, NODE_SIZE, True)
    esum = _sc_gather_sums(edge_repr, eidx, EDGE_SIZE, False)

    wn_stack = jnp.stack([W[:NODE_SIZE] for W in
                          (W_deg1, W_deg2, W_deg4, W_deg8)])
    we_stack = jnp.stack([W[NODE_SIZE:] for W in
                          (W_deg1, W_deg2, W_deg4, W_deg8)])
    return _tc_fused(node_repr, nsum, esum, W_self, wn_stack, we_stack, bias)


# speedup vs baseline: 1.1467x; 1.1254x over previous
import functools

import jax
import jax.numpy as jnp
from jax import lax
from jax.experimental import pallas as pl
from jax.experimental.pallas import tpu as pltpu
from jax.experimental.pallas import tpu_sc as plsc

N_NODES = 100000
NODE_SIZE = 128
EDGE_SIZE = 16
OUT_SIZE = 128
GROUP = 25000
DEGS = (1, 2, 4, 8)
N_EDGES = 375000
EPS = 1e-5

NC = 2
NS = 16
NW = NC * NS
LANES = 16
BLK = 128


def _ceil_div(a, b):
    return (a + b - 1) // b


NCHUNK = _ceil_div(_ceil_div(GROUP, BLK), NW)
SPAN = NCHUNK * BLK


def _make_sc_gather_body(width):
    def body(table_hbm, i1, i2, i3, i4, sum_hbm,
             idx_v, rows, accb, gsem, osem):
        wid = lax.axis_index("s") * NC + lax.axis_index("c")
        idx_all = (i1, i2, i3, i4)
        s0 = pl.multiple_of(jnp.minimum(wid * SPAN, GROUP - SPAN), 8)
        for b, d in enumerate(DEGS):
            idx_hbm = idx_all[b]
            for k in range(d):
                pltpu.sync_copy(idx_hbm.at[pl.ds(k * GROUP + s0, SPAN)],
                                idx_v.at[pl.ds(k * SPAN, SPAN)])

            steps = [(j, k) for j in range(NCHUNK) for k in range(d)]
            nst = len(steps)

            def issue_g(i, p):
                j, k = steps[i]
                pltpu.async_copy(
                    table_hbm.at[idx_v.at[pl.ds(k * SPAN + j * BLK, BLK)]],
                    rows.at[p], gsem.at[p])

            def wait_g(p):
                pltpu.make_async_copy(
                    table_hbm.at[idx_v.at[pl.ds(0, BLK)]],
                    rows.at[p], gsem.at[p]).wait()

            def acc(p, q, first):
                if first:
                    def cp_row(r, _):
                        for c in range(width // LANES):
                            accb[q, r, pl.ds(c * LANES, LANES)] = (
                                rows[p, r, pl.ds(c * LANES, LANES)])
                        return 0
                    lax.fori_loop(0, BLK, cp_row, 0, unroll=2)
                else:
                    def add_row(r, _):
                        for c in range(width // LANES):
                            accb[q, r, pl.ds(c * LANES, LANES)] += (
                                rows[p, r, pl.ds(c * LANES, LANES)])
                        return 0
                    lax.fori_loop(0, BLK, add_row, 0, unroll=1)

            def issue_o(j, q, b=b):
                obase = pl.multiple_of(s0 + j * BLK, 8)
                pltpu.async_copy(
                    accb.at[q, pl.ds(0, BLK)],
                    sum_hbm.at[b, pl.ds(obase, BLK)], osem.at[q])

            def wait_o(q, b=b):
                pltpu.make_async_copy(
                    accb.at[q, pl.ds(0, BLK)],
                    sum_hbm.at[b, pl.ds(0, BLK)], osem.at[q])  .wait()

            issue_g(0, 0)
            issue_g(1, 1)
            for i, (j, k) in enumerate(steps):
                p = i % 2
                q = j % 2
                wait_g(p)
                if k == 0 and j >= 2:
                    wait_o(q)
                acc(p, q, k == 0)
                if i + 2 < nst:
                    issue_g(i + 2, p)
                if k == d - 1:
                    issue_o(j, q)
            wait_o(0)
            wait_o(1)

    return body


def _sc_gather_sums(table, idx4, width, tc_tiling):
    mesh = plsc.VectorSubcoreMesh(core_axis_name="c", subcore_axis_name="s")
    out_type = jax.ShapeDtypeStruct((4, GROUP, width), jnp.float32)
    scratch = [
        pltpu.VMEM((max(DEGS) * SPAN,), jnp.int32),
        pltpu.VMEM((2, BLK, width), jnp.float32),
        pltpu.VMEM((2, BLK, width), jnp.float32),
        pltpu.SemaphoreType.DMA((2,)),
        pltpu.SemaphoreType.DMA((2,)),
    ]
    fn = pl.kernel(_make_sc_gather_body(width), out_type=out_type, mesh=mesh,
                   scratch_types=scratch,
                   compiler_params=pltpu.CompilerParams(
                       use_tc_tiling_on_sc=tc_tiling))
    return fn(table, *idx4)


R1 = 1000
NB1 = 4 * GROUP // R1
BPB = GROUP // R1


def _tc_fused_body(node_ref, nsum_ref, esum_ref, wself_ref, wn_ref, we_ref,
                   bias_ref, out_ref, act_scr, s1_scr, s2_scr, stat_scr):
    p = pl.program_id(0)
    g = pl.program_id(1)

    @pl.when(p == 0)
    def _():
        act = jnp.dot(node_ref[...], wself_ref[...],
                      preferred_element_type=jnp.float32) + bias_ref[...]
        act = act + jnp.dot(nsum_ref[0], wn_ref[0],
                            preferred_element_type=jnp.float32)
        act = act + jnp.dot(esum_ref[0], we_ref[0],
                            preferred_element_type=jnp.float32)
        act_scr[pl.ds(g * R1, R1), :] = act
        m1 = jnp.sum(act.reshape(R1 // 8, 8, OUT_SIZE), axis=0)
        m2 = jnp.sum((act * act).reshape(R1 // 8, 8, OUT_SIZE), axis=0)

        @pl.when(g == 0)
        def _():
            s1_scr[...] = m1
            s2_scr[...] = m2

        @pl.when(g > 0)
        def _():
            s1_scr[...] += m1
            s2_scr[...] += m2

    @pl.when(p == 1)
    def _():
        @pl.when(g == 0)
        def _():
            n = jnp.float32(4 * GROUP)
            mu = jnp.sum(s1_scr[...], axis=0, keepdims=True) / n
            var = jnp.sum(s2_scr[...], axis=0, keepdims=True) / n - mu * mu
            stat_scr[0:1, :] = mu
            stat_scr[1:2, :] = lax.rsqrt(var + EPS)

        act = act_scr[pl.ds(g * R1, R1), :]
        out_ref[...] = jnp.maximum(
            (act - stat_scr[0:1, :]) * stat_scr[1:2, :], 0.0)


def _tc_fused(node_repr, nsum, esum, W_self, wn_stack, we_stack, bias):
    return pl.pallas_call(
        _tc_fused_body,
        grid=(2, NB1),
        in_specs=[
            pl.BlockSpec((R1, NODE_SIZE),
                         lambda p, g: (jnp.where(p == 0, g, NB1 - 1), 0)),
            pl.BlockSpec((1, R1, NODE_SIZE),
                         lambda p, g: (jnp.where(p == 0, g // BPB, 3),
                                       jnp.where(p == 0, g % BPB, BPB - 1),
                                       0)),
            pl.BlockSpec((1, R1, EDGE_SIZE),
                         lambda p, g: (jnp.where(p == 0, g // BPB, 3),
                                       jnp.where(p == 0, g % BPB, BPB - 1),
                                       0)),
            pl.BlockSpec((NODE_SIZE, OUT_SIZE), lambda p, g: (0, 0)),
            pl.BlockSpec((1, NODE_SIZE, OUT_SIZE),
                         lambda p, g: (jnp.where(p == 0, g // BPB, 3), 0, 0)),
            pl.BlockSpec((1, EDGE_SIZE, OUT_SIZE),
                         lambda p, g: (jnp.where(p == 0, g // BPB, 3), 0, 0)),
            pl.BlockSpec((1, OUT_SIZE), lambda p, g: (0, 0)),
        ],
        out_specs=pl.BlockSpec((R1, OUT_SIZE),
                               lambda p, g: (jnp.where(p == 1, g, 0), 0)),
        out_shape=jax.ShapeDtypeStruct((4 * GROUP, OUT_SIZE), jnp.float32),
        scratch_shapes=[
            pltpu.VMEM((4 * GROUP, OUT_SIZE), jnp.float32),
            pltpu.VMEM((8, OUT_SIZE), jnp.float32),
            pltpu.VMEM((8, OUT_SIZE), jnp.float32),
            pltpu.VMEM((8, OUT_SIZE), jnp.float32),
        ],
        compiler_params=pltpu.CompilerParams(
            vmem_limit_bytes=128 * 1024 * 1024),
    )(node_repr, nsum, esum, W_self, wn_stack, we_stack, bias)


def kernel(node_repr, edge_repr, nb_node_d1, nb_edge_d1, nb_node_d2,
           nb_edge_d2, nb_node_d4, nb_edge_d4, nb_node_d8, nb_edge_d8,
           W_self, W_deg1, W_deg2, W_deg4, W_deg8, W_deg16, bias):
    nidx = tuple(a.T.reshape(-1).astype(jnp.int32)
                 for a in (nb_node_d1, nb_node_d2, nb_node_d4, nb_node_d8))
    eidx = tuple(a.T.reshape(-1).astype(jnp.int32)
                 for a in (nb_edge_d1, nb_edge_d2, nb_edge_d4, nb_edge_d8))
    nsum = _sc_gather_sums(node_repr, nidx, NODE_SIZE, True)
    eidx, nsum = lax.optimization_barrier((eidx, nsum))
    esum = _sc_gather_sums(edge_repr, eidx, EDGE_SIZE, False)

    wn_stack = jnp.stack([W[:NODE_SIZE] for W in
                          (W_deg1, W_deg2, W_deg4, W_deg8)])
    we_stack = jnp.stack([W[NODE_SIZE:] for W in
                          (W_deg1, W_deg2, W_deg4, W_deg8)])
    return _tc_fused(node_repr, nsum, esum, W_self, wn_stack, we_stack, bias)
